# baseline (speedup 1.0000x reference)
import jax
import jax.numpy as jnp
from jax import lax
from jax.experimental import pallas as pl
from jax.experimental.pallas import tpu as pltpu
from jax.experimental.pallas import tpu_sc as plsc

VOCAB = 1000000
N_EMBD = 64
SEQ = 200
BATCH = 4096

NC, NS = 2, 16
NW = NC * NS
NCOLS = (VOCAB + 127) // 128
K1_PER_W = 245
K1_PAIRS = 123
NPAIR = VOCAB // 2 + 32
NT8 = SEQ // 8
NBC = BATCH // 128


def _iota16():
    return lax.iota(jnp.int32, 16)


def _splat16(v):
    return jnp.zeros((16,), jnp.int32) + v


def _k1_body(ttok, ttail, out,
             in_a, in_b, out_a, out_b, isem_a, isem_b, wsem_a, wsem_b):
    wid = lax.axis_index("s") * NC + lax.axis_index("c")
    base = wid * K1_PER_W
    last = NCOLS - 1

    def start_in(col, buf, sem):
        dst = buf.at[:, pl.ds(0, 128)]

        @pl.when(col < last)
        def _full():
            pltpu.async_copy(ttok.at[:, pl.ds(col * 128, 128)], dst, sem)

        @pl.when(col == last)
        def _tail():
            pltpu.async_copy(ttail, dst, sem)

    def wait_in(buf, sem):
        pltpu.make_async_copy(ttok.at[:, pl.ds(0, 128)],
                              buf.at[:, pl.ds(0, 128)], sem).wait()

    def transpose(in_v, out_v):
        @plsc.parallel_loop(0, 8, unroll=4)
        def g_body(g):
            for pp in range(8):
                p = g * 8 + pp
                for k in range(8):
                    l = 2 * p + (1 if k >= 4 else 0)
                    e0 = 16 * (k % 4)
                    x = plsc.load_gather(in_v, [_iota16() + e0, _splat16(l)])
                    out_v[p, pl.ds(e0 + (64 if k >= 4 else 0), 16)] = x

    def write_out(col, out_v, wsem):
        pltpu.async_copy(out_v, out.at[pl.ds(col * 64, 64)], wsem)

    def drain_out(out_v, wsem):
        pltpu.make_async_copy(out_v, out.at[pl.ds(0, 64)], wsem).wait()

    def ce(i):
        return jnp.minimum(base + i, last)

    start_in(ce(0), in_a, isem_a)

    def pair_body(i, carry):
        ca = ce(2 * i)
        cb = ce(2 * i + 1)
        cn = ce(2 * i + 2)
        wait_in(in_a, isem_a)
        start_in(cb, in_b, isem_b)

        @pl.when(i > 0)
        def _da():
            drain_out(out_a, wsem_a)

        transpose(in_a, out_a)
        write_out(ca, out_a, wsem_a)
        wait_in(in_b, isem_b)

        @pl.when(i < K1_PAIRS - 1)
        def _na():
            start_in(cn, in_a, isem_a)

        @pl.when(i > 0)
        def _db():
            drain_out(out_b, wsem_b)

        transpose(in_b, out_b)
        write_out(cb, out_b, wsem_b)
        return carry

    lax.fori_loop(0, K1_PAIRS, pair_body, 0)
    drain_out(out_a, wsem_a)
    drain_out(out_b, wsem_b)


def _k2_body(xt, posp, table, out,
             idx_v, idx2_v, par_v, pos_v, pos_t, rows_a, rows_b,
             out_a, out_b, gsem_a, gsem_b, wsem_a, wsem_b):
    wid = lax.axis_index("s") * NC + lax.axis_index("c")
    bc = wid

    def start_g(tl, rows, sem):
        pltpu.async_copy(table.at[idx2_v.at[tl]],
                         rows.at[:, pl.ds(0, 128)], sem)

    def wait_g(rows, sem):
        pltpu.make_async_copy(table.at[idx2_v.at[0]],
                              rows.at[:, pl.ds(0, 128)], sem).wait()

    def proc(tl, rows_v, out_v):
        for k in range(4):
            pc = pos_v[tl, pl.ds(16 * k, 16)]
            evec = _iota16() + 16 * k
            for c in range(16):
                plsc.store_scatter(pos_t, [evec, _splat16(c)], pc)

        @plsc.parallel_loop(0, 8, unroll=4)
        def m_body(m):
            rvec = _iota16() + 16 * m
            par = par_v[tl, pl.ds(m * 16, 16)]
            for e in range(64):
                x = plsc.load_gather(rows_v, [rvec, par + e])
                out_v[e, pl.ds(m * 16, 16)] = x + pos_t[e, pl.ds(0, 16)]

    def write_o(t_abs, out_v, wsem):
        for er in range(8):
            pltpu.async_copy(out_v.at[pl.ds(er * 8, 8), pl.ds(0, 128)],
                             out.at[t_abs, er, bc], wsem)

    def drain_w(out_v, wsem):
        for er in range(8):
            pltpu.make_async_copy(out_v.at[pl.ds(0, 8), pl.ds(0, 128)],
                                  out.at[0, 0, bc], wsem).wait()

    def block(t8, carry):
        pltpu.sync_copy(xt.at[pl.ds(t8 * 8, 8), pl.ds(bc * 128, 128)], idx_v)
        pltpu.sync_copy(posp.at[pl.ds(t8 * 8, 8)], pos_v)
        for r in range(8):
            for m in range(8):
                v = idx_v[r, pl.ds(16 * m, 16)]
                idx2_v[r, pl.ds(16 * m, 16)] = lax.shift_right_logical(v, 1)
                par_v[r, pl.ds(16 * m, 16)] = lax.shift_left(
                    lax.bitwise_and(v, 1), 6)
        start_g(0, rows_a, gsem_a)

        def tp_body(tp, c2):
            ta = 2 * tp
            tb = 2 * tp + 1
            not_first = jnp.logical_or(t8 > 0, tp > 0)
            wait_g(rows_a, gsem_a)
            start_g(tb, rows_b, gsem_b)

            @pl.when(not_first)
            def _da():
                drain_w(out_a, wsem_a)

            proc(ta, rows_a, out_a)
            write_o(t8 * 8 + ta, out_a, wsem_a)
            wait_g(rows_b, gsem_b)

            @pl.when(tp < 3)
            def _ng():
                start_g(ta + 2, rows_a, gsem_a)

            @pl.when(not_first)
            def _db():
                drain_w(out_b, wsem_b)

            proc(tb, rows_b, out_b)
            write_o(t8 * 8 + tb, out_b, wsem_b)
            return c2

        lax.fori_loop(0, 4, tp_body, 0)
        return carry

    lax.fori_loop(0, NT8, block, 0)
    drain_w(out_a, wsem_a)
    drain_w(out_b, wsem_b)


def kernel(x, tok_emb, pos_emb):
    mesh = plsc.VectorSubcoreMesh(core_axis_name="c", subcore_axis_name="s")
    params = pltpu.CompilerParams(use_tc_tiling_on_sc=True,
                                  needs_layout_passes=False)

    k1 = pl.kernel(
        _k1_body,
        out_type=jax.ShapeDtypeStruct((NPAIR, 128), jnp.float32),
        mesh=mesh,
        compiler_params=params,
        scratch_types=[
            pltpu.VMEM((64, 129), jnp.float32),
            pltpu.VMEM((64, 129), jnp.float32),
            pltpu.VMEM((64, 128), jnp.float32),
            pltpu.VMEM((64, 128), jnp.float32),
            pltpu.SemaphoreType.DMA,
            pltpu.SemaphoreType.DMA,
            pltpu.SemaphoreType.DMA,
            pltpu.SemaphoreType.DMA,
        ],
    )
    k2 = pl.kernel(
        _k2_body,
        out_type=jax.ShapeDtypeStruct((SEQ, 8, NBC, 8, 128), jnp.float32),
        mesh=mesh,
        compiler_params=params,
        scratch_types=[
            pltpu.VMEM((8, 128), jnp.int32),
            pltpu.VMEM((8, 128), jnp.int32),
            pltpu.VMEM((8, 128), jnp.int32),
            pltpu.VMEM((8, 128), jnp.float32),
            pltpu.VMEM((64, 17), jnp.float32),
            pltpu.VMEM((128, 129), jnp.float32),
            pltpu.VMEM((128, 129), jnp.float32),
            pltpu.VMEM((64, 128), jnp.float32),
            pltpu.VMEM((64, 128), jnp.float32),
            pltpu.SemaphoreType.DMA,
            pltpu.SemaphoreType.DMA,
            pltpu.SemaphoreType.DMA,
            pltpu.SemaphoreType.DMA,
        ],
    )

    ttok = tok_emb.T
    ttail = jnp.pad(tok_emb[VOCAB - 64:].T, ((0, 0), (0, 64)))
    table = k1(ttok, ttail)
    xt = x.astype(jnp.int32).T
    posp = jnp.pad(pos_emb[:SEQ], ((0, 0), (0, 64)))
    o5 = k2(xt, posp, table)
    return o5.transpose(2, 4, 0, 1, 3).reshape(BATCH, SEQ, N_EMBD)


# speedup vs baseline: 1.4381x; 1.4381x over previous
import jax
import jax.numpy as jnp
from jax import lax
from jax.experimental import pallas as pl
from jax.experimental.pallas import tpu as pltpu
from jax.experimental.pallas import tpu_sc as plsc

N_EMBD = 64
SEQ = 200
BATCH = 4096

NC, NS = 2, 16
NW = NC * NS
TOTAL = BATCH * SEQ
RPW = TOTAL // NW
CHUNK = 400
NCHUNK = RPW // CHUNK
NSTREAM = 5
SPS = CHUNK // NSTREAM
LPR = N_EMBD // 16


def _body(x_hbm, pos_hbm, tok_hbm, out_hbm, idx_v, rows_v, pos_v, gsem):
    cid = lax.axis_index("c")
    sid = lax.axis_index("s")
    wid = sid * NC + cid
    pltpu.sync_copy(pos_hbm, pos_v)
    base = wid * RPW

    def chunk_body(c, carry):
        pltpu.sync_copy(x_hbm.at[wid * NCHUNK + c], idx_v)
        descs = []
        for j in range(NSTREAM):
            descs.append(pltpu.async_copy(
                tok_hbm.at[idx_v.at[j]],
                rows_v.at[pl.ds(j * SPS, SPS)], gsem))
        for d in descs:
            d.wait()

        def add_body(r, carry2):
            for k in range(LPR):
                plsc.addupdate(rows_v.at[r, pl.ds(k * 16, 16)],
                               pos_v[r, pl.ds(k * 16, 16)])
            return carry2

        lax.fori_loop(0, CHUNK, add_body, 0, unroll=4)
        pltpu.sync_copy(rows_v, out_hbm.at[pl.ds(base + c * CHUNK, CHUNK)])
        return carry

    lax.fori_loop(0, NCHUNK, chunk_body, 0)


def kernel(x, tok_emb, pos_emb):
    x2 = x.astype(jnp.int32).reshape(NW * NCHUNK, NSTREAM, SPS)
    pos_rep = jnp.concatenate([pos_emb[:SEQ]] * (CHUNK // SEQ), axis=0)
    mesh = plsc.VectorSubcoreMesh(core_axis_name="c", subcore_axis_name="s")
    f = pl.kernel(
        _body,
        out_type=jax.ShapeDtypeStruct((TOTAL, N_EMBD), jnp.float32),
        mesh=mesh,
        compiler_params=pltpu.CompilerParams(use_tc_tiling_on_sc=False),
        scratch_types=[
            pltpu.VMEM((NSTREAM, SPS), jnp.int32),
            pltpu.VMEM((CHUNK, N_EMBD), jnp.float32),
            pltpu.VMEM((CHUNK, N_EMBD), jnp.float32),
            pltpu.SemaphoreType.DMA,
        ],
    )
    out = f(x2, pos_rep, tok_emb)
    return out.reshape(BATCH, SEQ, N_EMBD)


# speedup vs baseline: 1.5537x; 1.0803x over previous
import jax
import jax.numpy as jnp
from jax import lax
from jax.experimental import pallas as pl
from jax.experimental.pallas import tpu as pltpu
from jax.experimental.pallas import tpu_sc as plsc

N_EMBD = 64
SEQ = 200
BATCH = 4096

NC, NS = 2, 16
NW = NC * NS
TOTAL = BATCH * SEQ
RPW = TOTAL // NW
CHUNK = 400
NCHUNK = RPW // CHUNK
NSTREAM = 5
SPS = CHUNK // NSTREAM
LPR = N_EMBD // 16


def _body(x_hbm, pos_hbm, tok_hbm, out_hbm,
          idx_a, idx_b, rows_a, rows_b, pos_v, gsem_a, gsem_b,
          wsem_a, wsem_b):
    cid = lax.axis_index("c")
    sid = lax.axis_index("s")
    wid = sid * NC + cid
    pltpu.sync_copy(pos_hbm, pos_v)
    base = wid * RPW

    def start_gathers(c, idx_v, rows_v, gsem):
        pltpu.sync_copy(x_hbm.at[wid * NCHUNK + c], idx_v)
        for j in range(NSTREAM):
            pltpu.async_copy(tok_hbm.at[idx_v.at[j]],
                             rows_v.at[pl.ds(j * SPS, SPS)], gsem)

    def wait_gathers(idx_v, rows_v, gsem):
        for j in range(NSTREAM):
            pltpu.make_async_copy(tok_hbm.at[idx_v.at[j]],
                                  rows_v.at[pl.ds(j * SPS, SPS)],
                                  gsem).wait()

    def add_pos(rows_v):
        def add_body(r, carry2):
            for k in range(LPR):
                plsc.addupdate(rows_v.at[r, pl.ds(k * 16, 16)],
                               pos_v[r, pl.ds(k * 16, 16)])
            return carry2

        lax.fori_loop(0, CHUNK, add_body, 0, unroll=4)

    def write_out(c, rows_v, wsem):
        pltpu.async_copy(rows_v, out_hbm.at[pl.ds(base + c * CHUNK, CHUNK)],
                         wsem)

    def drain_out(rows_v, wsem):
        pltpu.make_async_copy(rows_v, out_hbm.at[pl.ds(base, CHUNK)],
                              wsem).wait()

    start_gathers(0, idx_a, rows_a, gsem_a)

    def pair_body(i, carry):
        ca = 2 * i
        cb = 2 * i + 1
        wait_gathers(idx_a, rows_a, gsem_a)

        @pl.when(i > 0)
        def _db():
            drain_out(rows_b, wsem_b)

        start_gathers(cb, idx_b, rows_b, gsem_b)
        add_pos(rows_a)
        write_out(ca, rows_a, wsem_a)
        wait_gathers(idx_b, rows_b, gsem_b)
        drain_out(rows_a, wsem_a)

        @pl.when(i < NCHUNK // 2 - 1)
        def _na():
            start_gathers(ca + 2, idx_a, rows_a, gsem_a)

        add_pos(rows_b)
        write_out(cb, rows_b, wsem_b)
        return carry

    lax.fori_loop(0, NCHUNK // 2, pair_body, 0)
    drain_out(rows_b, wsem_b)


def kernel(x, tok_emb, pos_emb):
    x2 = x.astype(jnp.int32).reshape(NW * NCHUNK, NSTREAM, SPS)
    pos_rep = jnp.concatenate([pos_emb[:SEQ]] * (CHUNK // SEQ), axis=0)
    mesh = plsc.VectorSubcoreMesh(core_axis_name="c", subcore_axis_name="s")
    f = pl.kernel(
        _body,
        out_type=jax.ShapeDtypeStruct((TOTAL, N_EMBD), jnp.float32),
        mesh=mesh,
        compiler_params=pltpu.CompilerParams(use_tc_tiling_on_sc=False),
        scratch_types=[
            pltpu.VMEM((NSTREAM, SPS), jnp.int32),
            pltpu.VMEM((NSTREAM, SPS), jnp.int32),
            pltpu.VMEM((CHUNK, N_EMBD), jnp.float32),
            pltpu.VMEM((CHUNK, N_EMBD), jnp.float32),
            pltpu.VMEM((CHUNK, N_EMBD), jnp.float32),
            pltpu.SemaphoreType.DMA,
            pltpu.SemaphoreType.DMA,
            pltpu.SemaphoreType.DMA,
            pltpu.SemaphoreType.DMA,
        ],
    )
    out = f(x2, pos_rep, tok_emb)
    return out.reshape(BATCH, SEQ, N_EMBD)
